# Initial kernel scaffold; baseline (speedup 1.0000x reference)
import jax
import jax.numpy as jnp
from jax.experimental import pallas as pl


def kernel(user_id, tag_input_ids, movie_id, genre_input_ids, user_table, movie_table, tag_table, genre_table, uW1, uB1, uW2, uB2, iW1, iB1, iW2, iB2):
    raise NotImplementedError("write your pallas kernel here")


# speedup vs baseline: 1.4561x; 1.4561x over previous
import functools

import jax
import jax.numpy as jnp
from jax import lax
from jax.experimental import pallas as pl
from jax.experimental.pallas import tpu as pltpu
from jax.experimental.pallas import tpu_sc as plsc

B = 4096
LT = 50
LG = 20
D = 32
NC = 2
NS = 16
NW = NC * NS
BPW = B // NW

_MESH = plsc.VectorSubcoreMesh(
    core_axis_name="c", subcore_axis_name="s", num_cores=NC, num_subcores=NS
)


def _sc_gather_body(uid_hbm, mid_hbm, tagT_hbm, genT_hbm,
                    user_tab, movie_tab, tag_tab, gen_tab,
                    u_out, m_out, tsum_out, gsum_out,
                    idx_v, rows_v, acc_v, tidx_v, gidx_v, sem):
    wid = lax.axis_index("s") * NC + lax.axis_index("c")
    base = wid * BPW

    pltpu.sync_copy(uid_hbm.at[pl.ds(base, BPW)], idx_v)
    pltpu.async_copy(user_tab.at[idx_v], rows_v, sem).wait()
    pltpu.sync_copy(rows_v, u_out.at[pl.ds(base, BPW)])

    pltpu.sync_copy(mid_hbm.at[pl.ds(base, BPW)], idx_v)
    pltpu.async_copy(movie_tab.at[idx_v], rows_v, sem).wait()
    pltpu.sync_copy(rows_v, m_out.at[pl.ds(base, BPW)])

    pltpu.sync_copy(tagT_hbm.at[:, pl.ds(base, BPW)], tidx_v)
    pltpu.async_copy(tag_tab.at[tidx_v.at[0]], acc_v, sem).wait()

    @pl.loop(1, LT)
    def _tag_step(l):
        pltpu.async_copy(tag_tab.at[tidx_v.at[l]], rows_v, sem).wait()

        @pl.loop(0, BPW, unroll=8)
        def _acc(r):
            plsc.addupdate(acc_v.at[r, pl.ds(0, 16)], rows_v[r, pl.ds(0, 16)])
            plsc.addupdate(acc_v.at[r, pl.ds(16, 16)], rows_v[r, pl.ds(16, 16)])

    pltpu.sync_copy(acc_v, tsum_out.at[pl.ds(base, BPW)])

    pltpu.sync_copy(genT_hbm.at[:, pl.ds(base, BPW)], gidx_v)
    pltpu.async_copy(gen_tab.at[gidx_v.at[0]], acc_v, sem).wait()

    @pl.loop(1, LG)
    def _gen_step(l):
        pltpu.async_copy(gen_tab.at[gidx_v.at[l]], rows_v, sem).wait()

        @pl.loop(0, BPW, unroll=8)
        def _acc(r):
            plsc.addupdate(acc_v.at[r, pl.ds(0, 16)], rows_v[r, pl.ds(0, 16)])
            plsc.addupdate(acc_v.at[r, pl.ds(16, 16)], rows_v[r, pl.ds(16, 16)])

    pltpu.sync_copy(acc_v, gsum_out.at[pl.ds(base, BPW)])


@jax.jit
def _sc_gather(user_id, movie_id, tagT, genT,
               user_table, movie_table, tag_table, genre_table):
    f32 = jnp.float32
    return pl.kernel(
        _sc_gather_body,
        out_type=[
            jax.ShapeDtypeStruct((B, D), f32),
            jax.ShapeDtypeStruct((B, D), f32),
            jax.ShapeDtypeStruct((B, D), f32),
            jax.ShapeDtypeStruct((B, D), f32),
        ],
        mesh=_MESH,
        compiler_params=pltpu.CompilerParams(use_tc_tiling_on_sc=False),
        scratch_types=[
            pltpu.VMEM((BPW,), jnp.int32),
            pltpu.VMEM((BPW, D), f32),
            pltpu.VMEM((BPW, D), f32),
            pltpu.VMEM((LT, BPW), jnp.int32),
            pltpu.VMEM((LG, BPW), jnp.int32),
            pltpu.SemaphoreType.DMA,
        ],
    )(user_id, movie_id, tagT, genT,
      user_table, movie_table, tag_table, genre_table)


def _tc_body(uemb_ref, tsum_ref, tag_ids_ref, iemb_ref, gsum_ref, gen_ids_ref,
             trow0_ref, grow0_ref,
             uW1_ref, uB1_ref, uW2_ref, uB2_ref,
             iW1_ref, iB1_ref, iW2_ref, iB2_ref, out_ref):
    def pooled(sum_ref, ids_ref, row0_ref, L):
        ids = ids_ref[...]
        nnz = jnp.sum((ids != 0).astype(jnp.float32), axis=1, keepdims=True)
        fallback = jnp.min(nnz) == 0.0
        denom = jnp.where(fallback, float(L), jnp.maximum(nnz, 1.0))
        eff = jnp.where(fallback, sum_ref[...],
                        sum_ref[...] - (float(L) - nnz) * row0_ref[...])
        return eff / denom

    tpool = pooled(tsum_ref, tag_ids_ref, trow0_ref, LT)
    gpool = pooled(gsum_ref, gen_ids_ref, grow0_ref, LG)

    def mlp(x, W1, b1, W2, b2):
        h = jnp.maximum(
            jnp.dot(x, W1, preferred_element_type=jnp.float32) + b1, 0.0)
        return jnp.dot(h, W2, preferred_element_type=jnp.float32) + b2

    user_in = jnp.concatenate([uemb_ref[...], tpool], axis=1)
    item_in = jnp.concatenate([iemb_ref[...], gpool], axis=1)
    uvec = mlp(user_in, uW1_ref[...], uB1_ref[...], uW2_ref[...], uB2_ref[...])
    ivec = mlp(item_in, iW1_ref[...], iB1_ref[...], iW2_ref[...], iB2_ref[...])
    out_ref[...] = jnp.sum(uvec * ivec, axis=1)


@jax.jit
def _tc_towers(uemb, tsum, tag_ids, iemb, gsum, gen_ids, trow0, grow0,
               uW1, uB1, uW2, uB2, iW1, iB1, iW2, iB2):
    return pl.pallas_call(
        _tc_body,
        out_shape=jax.ShapeDtypeStruct((B,), jnp.float32),
    )(uemb, tsum, tag_ids, iemb, gsum, gen_ids, trow0, grow0,
      uW1, uB1, uW2, uB2, iW1, iB1, iW2, iB2)


def kernel(user_id, tag_input_ids, movie_id, genre_input_ids,
           user_table, movie_table, tag_table, genre_table,
           uW1, uB1, uW2, uB2, iW1, iB1, iW2, iB2):
    user_id = user_id.astype(jnp.int32)
    movie_id = movie_id.astype(jnp.int32)
    tag_ids = tag_input_ids.astype(jnp.int32)
    gen_ids = genre_input_ids.astype(jnp.int32)

    uemb, iemb, tsum, gsum = _sc_gather(
        user_id, movie_id, tag_ids.T, gen_ids.T,
        user_table, movie_table, tag_table, genre_table)

    return _tc_towers(
        uemb, tsum, tag_ids, iemb, gsum, gen_ids,
        tag_table[0:1], genre_table[0:1],
        uW1, uB1.reshape(1, -1), uW2, uB2.reshape(1, -1),
        iW1, iB1.reshape(1, -1), iW2, iB2.reshape(1, -1))
